# baseline (speedup 1.0000x reference)
import functools

import jax
import jax.numpy as jnp
from jax import lax
from jax.experimental import pallas as pl
from jax.experimental.pallas import tpu as pltpu
from jax.experimental.pallas import tpu_sc as plsc

B = 16384
D = 128
EPS = 1e-5


try:
    _info = plsc.get_sparse_core_info()
    _NC = _info.num_cores
    _NS = _info.num_subcores
except Exception:
    _NC, _NS = 2, 16
_NW = _NC * _NS
_BPW = B // _NW
_CH = 128
_NCH = _BPW // _CH


def _sc_gather2(uidx, iidx, t0, t1):
    mesh = plsc.VectorSubcoreMesh(core_axis_name="c", subcore_axis_name="s")
    f32 = jnp.float32

    @functools.partial(
        pl.kernel,
        mesh=mesh,
        out_type=[jax.ShapeDtypeStruct((B, D), f32) for _ in range(2)],
        scratch_types=(
            [pltpu.VMEM((_CH,), jnp.int32) for _ in range(4)]
            + [pltpu.VMEM((_CH, D), f32) for _ in range(4)]
            + [pltpu.SemaphoreType.DMA for _ in range(4)]
        ),
    )
    def gather_k(uidx_h, iidx_h, t0_h, t1_h,
                 o0_h, o1_h,
                 uv0, uv1, iv0, iv1,
                 b00, b10, b01, b11,
                 g0, g1, s0, s1):
        uv = (uv0, uv1)
        iv = (iv0, iv1)
        ubuf = (b00, b01)
        ibuf = (b10, b11)
        gsem = (g0, g1)
        ssem = (s0, s1)
        wid = lax.axis_index("s") * _NC + lax.axis_index("c")
        base = wid * _BPW

        jobs = [(t0_h, t1_h, o0_h, o1_h, c) for c in range(_NCH)]
        nj = len(jobs)

        gh = [None] * nj
        sh = [None] * nj
        pltpu.sync_copy(uidx_h.at[pl.ds(base + jobs[0][4] * _CH, _CH)], uv[0])
        pltpu.sync_copy(iidx_h.at[pl.ds(base + jobs[0][4] * _CH, _CH)], iv[0])
        for j in range(nj):
            p = j % 2
            tu, ti, ou, oi, c = jobs[j]
            if j >= 2:
                sh[j - 2][0].wait()
                sh[j - 2][1].wait()
            gh[j] = (pltpu.async_copy(tu.at[uv[p]], ubuf[p], gsem[p]),
                     pltpu.async_copy(ti.at[iv[p]], ibuf[p], gsem[p]))
            if j >= 1:
                q = 1 - p
                tu_p, ti_p, ou_p, oi_p, c_p = jobs[j - 1]
                off_p = base + c_p * _CH
                gh[j - 1][0].wait()
                gh[j - 1][1].wait()
                sh[j - 1] = (
                    pltpu.async_copy(ubuf[q], ou_p.at[pl.ds(off_p, _CH)],
                                     ssem[q]),
                    pltpu.async_copy(ibuf[q], oi_p.at[pl.ds(off_p, _CH)],
                                     ssem[q]),
                )
            if j + 1 < nj:
                off_n = base + jobs[j + 1][4] * _CH
                pltpu.sync_copy(uidx_h.at[pl.ds(off_n, _CH)], uv[1 - p])
                pltpu.sync_copy(iidx_h.at[pl.ds(off_n, _CH)], iv[1 - p])
        j = nj - 1
        p = j % 2
        gh[j][0].wait()
        gh[j][1].wait()
        tu_p, ti_p, ou_p, oi_p, c_p = jobs[j]
        off_p = base + c_p * _CH
        sh[j] = (pltpu.async_copy(ubuf[p], ou_p.at[pl.ds(off_p, _CH)],
                                  ssem[p]),
                 pltpu.async_copy(ibuf[p], oi_p.at[pl.ds(off_p, _CH)],
                                  ssem[p]))
        sh[j - 1][0].wait()
        sh[j - 1][1].wait()
        sh[j][0].wait()
        sh[j][1].wait()

    return gather_k(uidx, iidx, t0, t1)


_BLK = 2048
_NB = B // _BLK


def _leaky(z):
    return jnp.where(z > 0, z, 0.1 * z)


def _accum_stats(a, st_ref):
    ps = jnp.stack([jnp.sum(a, axis=0), jnp.sum(a * a, axis=0)])

    @pl.when(pl.program_id(0) == 0)
    def _():
        st_ref[...] = ps

    @pl.when(pl.program_id(0) > 0)
    def _():
        st_ref[...] = st_ref[...] + ps


def _norm_params(st, g, be):
    m = st[0] * (1.0 / B)
    var = st[1] * (1.0 / B) - m * m
    scale = g * lax.rsqrt(var + EPS)
    shift = be - m * scale
    return scale, shift


def _stats_mxu(ab, st_ref):
    ones = jnp.ones((1, _BLK), jnp.bfloat16)
    su = jnp.dot(ones, ab, preferred_element_type=jnp.float32)
    ssq = jnp.dot(ones, ab * ab, preferred_element_type=jnp.float32)
    ps = jnp.concatenate([su, ssq], axis=0)

    @pl.when(pl.program_id(1) == 0)
    def _():
        st_ref[...] = ps

    @pl.when(pl.program_id(1) > 0)
    def _():
        st_ref[...] = st_ref[...] + ps


def _mega_body(um_ref, im_ref, ug_ref, ig_ref,
               w1_ref, b1_ref, g1_ref, be1_ref,
               w2_ref, b2_ref, g2_ref, be2_ref,
               w3_ref, b3_ref, g3_ref, be3_ref,
               wo_ref, bo_ref,
               o_ref,
               h1s, h2s, h3s, st1, st2, st3,
               w2p, b2p, w3p, b3p, wbp, csp):
    s = pl.program_id(0)
    i = pl.program_id(1)
    rows = pl.ds(i * _BLK, _BLK)
    bf16 = jnp.bfloat16
    f32 = jnp.float32

    @pl.when(s == 0)
    def _():
        w = w1_ref[...]
        z = (jnp.dot(um_ref[...].astype(bf16), w[:D],
                     preferred_element_type=f32)
             + jnp.dot(im_ref[...].astype(bf16), w[D:],
                       preferred_element_type=f32)
             + b1_ref[...])
        a = _leaky(z)
        ab = a.astype(bf16)
        h1s[rows, :] = ab
        _stats_mxu(ab, st1)

    @pl.when(s == 1)
    def _():
        @pl.when(i == 0)
        def _():
            scale, shift = _norm_params(st1[...], g1_ref[...], be1_ref[...])
            wf = w2_ref[...].astype(f32)
            w2p[...] = (scale[:, None] * wf).astype(bf16)
            b2p[...] = (jnp.dot(shift.astype(bf16)[None, :], w2_ref[...],
                                preferred_element_type=f32)
                        + b2_ref[...][None, :])

        z = (jnp.dot(h1s[rows, :], w2p[...], preferred_element_type=f32)
             + b2p[...])
        a = _leaky(z)
        ab = a.astype(bf16)
        h2s[rows, :] = ab
        _stats_mxu(ab, st2)

    @pl.when(s == 2)
    def _():
        @pl.when(i == 0)
        def _():
            scale, shift = _norm_params(st2[...], g2_ref[...], be2_ref[...])
            wf = w3_ref[...].astype(f32)
            w3p[...] = (scale[:, None] * wf).astype(bf16)
            b3p[...] = (jnp.dot(shift.astype(bf16)[None, :], w3_ref[...],
                                preferred_element_type=f32)
                        + b3_ref[...][None, :])

        z = (jnp.dot(h2s[rows, :], w3p[...], preferred_element_type=f32)
             + b3p[...])
        a = _leaky(z)
        ab = a.astype(bf16)
        h3s[rows, :] = ab
        _stats_mxu(ab, st3)

    @pl.when(s == 3)
    def _():
        wo = wo_ref[...][:, 0]

        @pl.when(i == 0)
        def _():
            scale, shift = _norm_params(st3[...], g3_ref[...], be3_ref[...])
            wbp[...] = (scale * wo[D:])[None, :]
            csp[...] = (shift * wo[D:])[None, :]

        gmf = ug_ref[...] * ig_ref[...]
        r = (jnp.sum(gmf * wo[:D] + h3s[rows, :].astype(f32) * wbp[...],
                     axis=1)
             + (jnp.sum(csp[...]) + bo_ref[0]))
        o_ref[...] = jax.nn.sigmoid(r)


def kernel(user_indices, item_indices, ue_gmf, ie_gmf, ue_mlp, ie_mlp,
           W1, b1, g1, be1, W2, b2, g2, be2, W3, b3, g3, be3, Wo, bo):
    uidx = user_indices.astype(jnp.int32)
    iidx = item_indices.astype(jnp.int32)

    um, im = _sc_gather2(uidx, iidx, ue_mlp, ie_mlp)
    ug, ig = _sc_gather2(uidx, iidx, ue_gmf, ie_gmf)

    f32 = jnp.float32
    bf16 = jnp.bfloat16

    def stage0_rows(h):
        return pl.BlockSpec(
            (_BLK, h), lambda s, i: (jnp.where(s == 0, i, 0), 0))

    def stage3_rows(h):
        return pl.BlockSpec(
            (_BLK, h), lambda s, i: (jnp.where(s == 3, i, 0), 0))

    def const2():
        return pl.BlockSpec(None, lambda s, i: (0, 0))

    def const1():
        return pl.BlockSpec(None, lambda s, i: (0,))

    out = pl.pallas_call(
        _mega_body,
        grid=(4, _NB),
        in_specs=[stage0_rows(D), stage0_rows(D),
                  stage3_rows(D), stage3_rows(D),
                  const2(), const1(), const1(), const1(),
                  const2(), const1(), const1(), const1(),
                  const2(), const1(), const1(), const1(),
                  const2(), const1()],
        out_specs=pl.BlockSpec((_BLK,), lambda s, i: (jnp.where(s == 3, i, 0),)),
        out_shape=jax.ShapeDtypeStruct((B,), f32),
        scratch_shapes=[
            pltpu.VMEM((B, 512), bf16),
            pltpu.VMEM((B, 256), bf16),
            pltpu.VMEM((B, 128), bf16),
            pltpu.VMEM((2, 512), f32),
            pltpu.VMEM((2, 256), f32),
            pltpu.VMEM((2, 128), f32),
            pltpu.VMEM((512, 256), bf16),
            pltpu.VMEM((1, 256), f32),
            pltpu.VMEM((256, 128), bf16),
            pltpu.VMEM((1, 128), f32),
            pltpu.VMEM((1, 128), f32),
            pltpu.VMEM((1, 128), f32),
        ],
    )(um, im, ug, ig,
      W1.astype(bf16), b1, g1, be1,
      W2.astype(bf16), b2, g2, be2,
      W3.astype(bf16), b3, g3, be3,
      Wo, bo)

    return out


# speedup vs baseline: 1.0662x; 1.0662x over previous
import functools

import jax
import jax.numpy as jnp
from jax import lax
from jax.experimental import pallas as pl
from jax.experimental.pallas import tpu as pltpu
from jax.experimental.pallas import tpu_sc as plsc

B = 16384
D = 128
EPS = 1e-5


try:
    _info = plsc.get_sparse_core_info()
    _NC = _info.num_cores
    _NS = _info.num_subcores
except Exception:
    _NC, _NS = 2, 16
_NW = _NC * _NS
_BPW = B // _NW
_CH = 128
_NCH = _BPW // _CH


def _sc_gather2(uidx, iidx, t0, t1):
    mesh = plsc.VectorSubcoreMesh(core_axis_name="c", subcore_axis_name="s")
    f32 = jnp.float32

    @functools.partial(
        pl.kernel,
        mesh=mesh,
        out_type=[jax.ShapeDtypeStruct((B, D), f32) for _ in range(2)],
        scratch_types=(
            [pltpu.VMEM((_CH,), jnp.int32) for _ in range(4)]
            + [pltpu.VMEM((_CH, D), f32) for _ in range(4)]
            + [pltpu.SemaphoreType.DMA for _ in range(4)]
        ),
    )
    def gather_k(uidx_h, iidx_h, t0_h, t1_h,
                 o0_h, o1_h,
                 uv0, uv1, iv0, iv1,
                 b00, b10, b01, b11,
                 g0, g1, s0, s1):
        uv = (uv0, uv1)
        iv = (iv0, iv1)
        ubuf = (b00, b01)
        ibuf = (b10, b11)
        gsem = (g0, g1)
        ssem = (s0, s1)
        wid = lax.axis_index("s") * _NC + lax.axis_index("c")
        base = wid * _BPW

        jobs = [(t0_h, t1_h, o0_h, o1_h, c) for c in range(_NCH)]
        nj = len(jobs)

        gh = [None] * nj
        sh = [None] * nj
        pltpu.sync_copy(uidx_h.at[pl.ds(base + jobs[0][4] * _CH, _CH)], uv[0])
        pltpu.sync_copy(iidx_h.at[pl.ds(base + jobs[0][4] * _CH, _CH)], iv[0])
        for j in range(nj):
            p = j % 2
            tu, ti, ou, oi, c = jobs[j]
            if j >= 2:
                sh[j - 2][0].wait()
                sh[j - 2][1].wait()
            gh[j] = (pltpu.async_copy(tu.at[uv[p]], ubuf[p], gsem[p]),
                     pltpu.async_copy(ti.at[iv[p]], ibuf[p], gsem[p]))
            if j >= 1:
                q = 1 - p
                tu_p, ti_p, ou_p, oi_p, c_p = jobs[j - 1]
                off_p = base + c_p * _CH
                gh[j - 1][0].wait()
                gh[j - 1][1].wait()
                sh[j - 1] = (
                    pltpu.async_copy(ubuf[q], ou_p.at[pl.ds(off_p, _CH)],
                                     ssem[q]),
                    pltpu.async_copy(ibuf[q], oi_p.at[pl.ds(off_p, _CH)],
                                     ssem[q]),
                )
            if j + 1 < nj:
                off_n = base + jobs[j + 1][4] * _CH
                pltpu.sync_copy(uidx_h.at[pl.ds(off_n, _CH)], uv[1 - p])
                pltpu.sync_copy(iidx_h.at[pl.ds(off_n, _CH)], iv[1 - p])
        j = nj - 1
        p = j % 2
        gh[j][0].wait()
        gh[j][1].wait()
        tu_p, ti_p, ou_p, oi_p, c_p = jobs[j]
        off_p = base + c_p * _CH
        sh[j] = (pltpu.async_copy(ubuf[p], ou_p.at[pl.ds(off_p, _CH)],
                                  ssem[p]),
                 pltpu.async_copy(ibuf[p], oi_p.at[pl.ds(off_p, _CH)],
                                  ssem[p]))
        sh[j - 1][0].wait()
        sh[j - 1][1].wait()
        sh[j][0].wait()
        sh[j][1].wait()

    return gather_k(uidx, iidx, t0, t1)


_BLK = 2048
_NB = B // _BLK


def _leaky(z):
    return jnp.where(z > 0, z, 0.1 * z)


def _accum_stats(a, st_ref):
    ps = jnp.stack([jnp.sum(a, axis=0), jnp.sum(a * a, axis=0)])

    @pl.when(pl.program_id(0) == 0)
    def _():
        st_ref[...] = ps

    @pl.when(pl.program_id(0) > 0)
    def _():
        st_ref[...] = st_ref[...] + ps


def _norm_params(st, g, be):
    m = st[0] * (1.0 / B)
    var = st[1] * (1.0 / B) - m * m
    scale = g * lax.rsqrt(var + EPS)
    shift = be - m * scale
    return scale, shift


def _stats_vpu(a, st_ref):
    ps = jnp.stack([jnp.sum(a, axis=0), jnp.sum(a * a, axis=0)])

    @pl.when(pl.program_id(1) == 0)
    def _():
        st_ref[...] = ps

    @pl.when(pl.program_id(1) > 0)
    def _():
        st_ref[...] = st_ref[...] + ps


def _mega_body(um_ref, im_ref, ug_ref, ig_ref,
               w1_ref, b1_ref, g1_ref, be1_ref,
               w2_ref, b2_ref, g2_ref, be2_ref,
               w3_ref, b3_ref, g3_ref, be3_ref,
               wo_ref, bo_ref,
               o_ref,
               h1s, h2s, h3s, st1, st2, st3,
               w2p, b2p, w3p, b3p, wbp, csp):
    s = pl.program_id(0)
    i = pl.program_id(1)
    rows = pl.ds(i * _BLK, _BLK)
    bf16 = jnp.bfloat16
    f32 = jnp.float32

    @pl.when(s == 0)
    def _():
        w = w1_ref[...]
        z = (jnp.dot(um_ref[...].astype(bf16), w[:D],
                     preferred_element_type=f32)
             + jnp.dot(im_ref[...].astype(bf16), w[D:],
                       preferred_element_type=f32)
             + b1_ref[...])
        a = _leaky(z)
        ab = a.astype(bf16)
        h1s[rows, :] = ab
        _stats_vpu(a, st1)

    @pl.when(s == 1)
    def _():
        @pl.when(i == 0)
        def _():
            scale, shift = _norm_params(st1[...], g1_ref[...], be1_ref[...])
            wf = w2_ref[...].astype(f32)
            w2p[...] = (scale[:, None] * wf).astype(bf16)
            b2p[...] = (jnp.dot(shift.astype(bf16)[None, :], w2_ref[...],
                                preferred_element_type=f32)
                        + b2_ref[...][None, :])

        z = (jnp.dot(h1s[rows, :], w2p[...], preferred_element_type=f32)
             + b2p[...])
        a = _leaky(z)
        ab = a.astype(bf16)
        h2s[rows, :] = ab
        _stats_vpu(a, st2)

    @pl.when(s == 2)
    def _():
        @pl.when(i == 0)
        def _():
            scale, shift = _norm_params(st2[...], g2_ref[...], be2_ref[...])
            wf = w3_ref[...].astype(f32)
            w3p[...] = (scale[:, None] * wf).astype(bf16)
            b3p[...] = (jnp.dot(shift.astype(bf16)[None, :], w3_ref[...],
                                preferred_element_type=f32)
                        + b3_ref[...][None, :])

        z = (jnp.dot(h2s[rows, :], w3p[...], preferred_element_type=f32)
             + b3p[...])
        a = _leaky(z)
        ab = a.astype(bf16)
        h3s[rows, :] = ab
        _stats_vpu(a, st3)

    @pl.when(s == 3)
    def _():
        wo = wo_ref[...][:, 0]

        @pl.when(i == 0)
        def _():
            scale, shift = _norm_params(st3[...], g3_ref[...], be3_ref[...])
            wbp[...] = (scale * wo[D:])[None, :]
            csp[...] = (shift * wo[D:])[None, :]

        gmf = ug_ref[...] * ig_ref[...]
        r = (jnp.sum(gmf * wo[:D] + h3s[rows, :].astype(f32) * wbp[...],
                     axis=1)
             + (jnp.sum(csp[...]) + bo_ref[0]))
        o_ref[...] = jax.nn.sigmoid(r)


def kernel(user_indices, item_indices, ue_gmf, ie_gmf, ue_mlp, ie_mlp,
           W1, b1, g1, be1, W2, b2, g2, be2, W3, b3, g3, be3, Wo, bo):
    uidx = user_indices.astype(jnp.int32)
    iidx = item_indices.astype(jnp.int32)

    um, im = _sc_gather2(uidx, iidx, ue_mlp, ie_mlp)
    ug, ig = _sc_gather2(uidx, iidx, ue_gmf, ie_gmf)

    f32 = jnp.float32
    bf16 = jnp.bfloat16

    def stage0_rows(h):
        return pl.BlockSpec(
            (_BLK, h), lambda s, i: (jnp.where(s == 0, i, 0), 0))

    def stage3_rows(h):
        return pl.BlockSpec(
            (_BLK, h), lambda s, i: (jnp.where(s == 3, i, 0), 0))

    def const2():
        return pl.BlockSpec(None, lambda s, i: (0, 0))

    def const1():
        return pl.BlockSpec(None, lambda s, i: (0,))

    out = pl.pallas_call(
        _mega_body,
        grid=(4, _NB),
        in_specs=[stage0_rows(D), stage0_rows(D),
                  stage3_rows(D), stage3_rows(D),
                  const2(), const1(), const1(), const1(),
                  const2(), const1(), const1(), const1(),
                  const2(), const1(), const1(), const1(),
                  const2(), const1()],
        out_specs=pl.BlockSpec((_BLK,), lambda s, i: (jnp.where(s == 3, i, 0),)),
        out_shape=jax.ShapeDtypeStruct((B,), f32),
        scratch_shapes=[
            pltpu.VMEM((B, 512), bf16),
            pltpu.VMEM((B, 256), bf16),
            pltpu.VMEM((B, 128), bf16),
            pltpu.VMEM((2, 512), f32),
            pltpu.VMEM((2, 256), f32),
            pltpu.VMEM((2, 128), f32),
            pltpu.VMEM((512, 256), bf16),
            pltpu.VMEM((1, 256), f32),
            pltpu.VMEM((256, 128), bf16),
            pltpu.VMEM((1, 128), f32),
            pltpu.VMEM((1, 128), f32),
            pltpu.VMEM((1, 128), f32),
        ],
    )(um, im, ug, ig,
      W1.astype(bf16), b1, g1, be1,
      W2.astype(bf16), b2, g2, be2,
      W3.astype(bf16), b3, g3, be3,
      Wo, bo)

    return out


# speedup vs baseline: 1.0755x; 1.0087x over previous
import functools

import jax
import jax.numpy as jnp
from jax import lax
from jax.experimental import pallas as pl
from jax.experimental.pallas import tpu as pltpu
from jax.experimental.pallas import tpu_sc as plsc

B = 16384
D = 128
EPS = 1e-5


try:
    _info = plsc.get_sparse_core_info()
    _NC = _info.num_cores
    _NS = _info.num_subcores
except Exception:
    _NC, _NS = 2, 16
_NW = _NC * _NS
_BPW = B // _NW
_CH = 128
_NCH = _BPW // _CH


def _sc_gather2(uidx, iidx, t0, t1):
    mesh = plsc.VectorSubcoreMesh(core_axis_name="c", subcore_axis_name="s")
    f32 = jnp.float32

    @functools.partial(
        pl.kernel,
        mesh=mesh,
        out_type=[jax.ShapeDtypeStruct((B, D), f32) for _ in range(2)],
        scratch_types=(
            [pltpu.VMEM((_CH,), jnp.int32) for _ in range(4)]
            + [pltpu.VMEM((_CH, D), f32) for _ in range(4)]
            + [pltpu.SemaphoreType.DMA for _ in range(4)]
        ),
    )
    def gather_k(uidx_h, iidx_h, t0_h, t1_h,
                 o0_h, o1_h,
                 uv0, uv1, iv0, iv1,
                 b00, b10, b01, b11,
                 g0, g1, s0, s1):
        uv = (uv0, uv1)
        iv = (iv0, iv1)
        ubuf = (b00, b01)
        ibuf = (b10, b11)
        gsem = (g0, g1)
        ssem = (s0, s1)
        wid = lax.axis_index("s") * _NC + lax.axis_index("c")
        base = wid * _BPW

        jobs = [(t0_h, t1_h, o0_h, o1_h, c) for c in range(_NCH)]
        nj = len(jobs)

        gh = [None] * nj
        sh = [None] * nj
        pltpu.sync_copy(uidx_h.at[pl.ds(base + jobs[0][4] * _CH, _CH)], uv[0])
        pltpu.sync_copy(iidx_h.at[pl.ds(base + jobs[0][4] * _CH, _CH)], iv[0])
        for j in range(nj):
            p = j % 2
            tu, ti, ou, oi, c = jobs[j]
            if j >= 2:
                sh[j - 2][0].wait()
                sh[j - 2][1].wait()
            gh[j] = (pltpu.async_copy(tu.at[uv[p]], ubuf[p], gsem[p]),
                     pltpu.async_copy(ti.at[iv[p]], ibuf[p], gsem[p]))
            if j >= 1:
                q = 1 - p
                tu_p, ti_p, ou_p, oi_p, c_p = jobs[j - 1]
                off_p = base + c_p * _CH
                gh[j - 1][0].wait()
                gh[j - 1][1].wait()
                sh[j - 1] = (
                    pltpu.async_copy(ubuf[q], ou_p.at[pl.ds(off_p, _CH)],
                                     ssem[q]),
                    pltpu.async_copy(ibuf[q], oi_p.at[pl.ds(off_p, _CH)],
                                     ssem[q]),
                )
            if j + 1 < nj:
                off_n = base + jobs[j + 1][4] * _CH
                pltpu.sync_copy(uidx_h.at[pl.ds(off_n, _CH)], uv[1 - p])
                pltpu.sync_copy(iidx_h.at[pl.ds(off_n, _CH)], iv[1 - p])
        j = nj - 1
        p = j % 2
        gh[j][0].wait()
        gh[j][1].wait()
        tu_p, ti_p, ou_p, oi_p, c_p = jobs[j]
        off_p = base + c_p * _CH
        sh[j] = (pltpu.async_copy(ubuf[p], ou_p.at[pl.ds(off_p, _CH)],
                                  ssem[p]),
                 pltpu.async_copy(ibuf[p], oi_p.at[pl.ds(off_p, _CH)],
                                  ssem[p]))
        sh[j - 1][0].wait()
        sh[j - 1][1].wait()
        sh[j][0].wait()
        sh[j][1].wait()

    return gather_k(uidx, iidx, t0, t1)


_BLK = 2048
_NB = B // _BLK


def _leaky(z):
    return jnp.maximum(z, 0.1 * z)


def _accum_stats(a, st_ref):
    ps = jnp.stack([jnp.sum(a, axis=0), jnp.sum(a * a, axis=0)])

    @pl.when(pl.program_id(0) == 0)
    def _():
        st_ref[...] = ps

    @pl.when(pl.program_id(0) > 0)
    def _():
        st_ref[...] = st_ref[...] + ps


def _norm_params(st, g, be):
    m = st[0] * (1.0 / B)
    var = st[1] * (1.0 / B) - m * m
    scale = g * lax.rsqrt(var + EPS)
    shift = be - m * scale
    return scale, shift


def _stats_vpu(a, st_ref):
    ps = jnp.stack([jnp.sum(a, axis=0), jnp.sum(a * a, axis=0)])

    @pl.when(pl.program_id(1) == 0)
    def _():
        st_ref[...] = ps

    @pl.when(pl.program_id(1) > 0)
    def _():
        st_ref[...] = st_ref[...] + ps


def _mega_body(um_ref, im_ref, ug_ref, ig_ref,
               w1_ref, b1_ref, g1_ref, be1_ref,
               w2_ref, b2_ref, g2_ref, be2_ref,
               w3_ref, b3_ref, g3_ref, be3_ref,
               wo_ref, bo_ref,
               o_ref,
               h1s, h2s, h3s, st1, st2, st3,
               w2p, b2p, w3p, b3p, wbp, csp):
    s = pl.program_id(0)
    i = pl.program_id(1)
    rows = pl.ds(i * _BLK, _BLK)
    bf16 = jnp.bfloat16
    f32 = jnp.float32

    @pl.when(s == 0)
    def _():
        w = w1_ref[...].astype(bf16)
        z = (jnp.dot(um_ref[...].astype(bf16), w[:D],
                     preferred_element_type=f32)
             + jnp.dot(im_ref[...].astype(bf16), w[D:],
                       preferred_element_type=f32)
             + b1_ref[...])
        a = _leaky(z)
        ab = a.astype(bf16)
        h1s[rows, :] = ab
        _stats_vpu(a, st1)

    @pl.when(s == 1)
    def _():
        @pl.when(i == 0)
        def _():
            scale, shift = _norm_params(st1[...], g1_ref[...], be1_ref[...])
            wf = w2_ref[...]
            w2p[...] = (scale[:, None] * wf).astype(bf16)
            b2p[...] = (jnp.dot(shift[None, :], wf,
                                preferred_element_type=f32)
                        + b2_ref[...][None, :])

        z = (jnp.dot(h1s[rows, :], w2p[...], preferred_element_type=f32)
             + b2p[...])
        a = _leaky(z)
        ab = a.astype(bf16)
        h2s[rows, :] = ab
        _stats_vpu(a, st2)

    @pl.when(s == 2)
    def _():
        @pl.when(i == 0)
        def _():
            scale, shift = _norm_params(st2[...], g2_ref[...], be2_ref[...])
            wf = w3_ref[...]
            w3p[...] = (scale[:, None] * wf).astype(bf16)
            b3p[...] = (jnp.dot(shift[None, :], wf,
                                preferred_element_type=f32)
                        + b3_ref[...][None, :])

        z = (jnp.dot(h2s[rows, :], w3p[...], preferred_element_type=f32)
             + b3p[...])
        a = _leaky(z)
        ab = a.astype(bf16)
        h3s[rows, :] = ab
        _stats_vpu(a, st3)

    @pl.when(s == 3)
    def _():
        wo = wo_ref[...][:, 0]

        @pl.when(i == 0)
        def _():
            scale, shift = _norm_params(st3[...], g3_ref[...], be3_ref[...])
            wbp[...] = (scale * wo[D:])[None, :]
            csp[...] = (shift * wo[D:])[None, :]

        gmf = ug_ref[...] * ig_ref[...]
        r = (jnp.sum(gmf * wo[:D] + h3s[rows, :].astype(f32) * wbp[...],
                     axis=1)
             + (jnp.sum(csp[...]) + bo_ref[0]))
        o_ref[...] = jax.nn.sigmoid(r)


def kernel(user_indices, item_indices, ue_gmf, ie_gmf, ue_mlp, ie_mlp,
           W1, b1, g1, be1, W2, b2, g2, be2, W3, b3, g3, be3, Wo, bo):
    uidx = user_indices.astype(jnp.int32)
    iidx = item_indices.astype(jnp.int32)

    um, im = _sc_gather2(uidx, iidx, ue_mlp, ie_mlp)
    ug, ig = _sc_gather2(uidx, iidx, ue_gmf, ie_gmf)

    f32 = jnp.float32
    bf16 = jnp.bfloat16

    def stage0_rows(h):
        return pl.BlockSpec(
            (_BLK, h), lambda s, i: (jnp.where(s == 0, i, 0), 0))

    def stage3_rows(h):
        return pl.BlockSpec(
            (_BLK, h), lambda s, i: (jnp.where(s == 3, i, 0), 0))

    def const2():
        return pl.BlockSpec(None, lambda s, i: (0, 0))

    def const1():
        return pl.BlockSpec(None, lambda s, i: (0,))

    out = pl.pallas_call(
        _mega_body,
        grid=(4, _NB),
        in_specs=[stage0_rows(D), stage0_rows(D),
                  stage3_rows(D), stage3_rows(D),
                  const2(), const1(), const1(), const1(),
                  const2(), const1(), const1(), const1(),
                  const2(), const1(), const1(), const1(),
                  const2(), const1()],
        out_specs=pl.BlockSpec((_BLK,), lambda s, i: (jnp.where(s == 3, i, 0),)),
        out_shape=jax.ShapeDtypeStruct((B,), f32),
        scratch_shapes=[
            pltpu.VMEM((B, 512), bf16),
            pltpu.VMEM((B, 256), bf16),
            pltpu.VMEM((B, 128), bf16),
            pltpu.VMEM((2, 512), f32),
            pltpu.VMEM((2, 256), f32),
            pltpu.VMEM((2, 128), f32),
            pltpu.VMEM((512, 256), bf16),
            pltpu.VMEM((1, 256), f32),
            pltpu.VMEM((256, 128), bf16),
            pltpu.VMEM((1, 128), f32),
            pltpu.VMEM((1, 128), f32),
            pltpu.VMEM((1, 128), f32),
        ],
    )(um, im, ug, ig,
      W1, b1, g1, be1,
      W2, b2, g2, be2,
      W3, b3, g3, be3,
      Wo, bo)

    return out


# speedup vs baseline: 1.1257x; 1.0467x over previous
import functools

import jax
import jax.numpy as jnp
from jax import lax
from jax.experimental import pallas as pl
from jax.experimental.pallas import tpu as pltpu
from jax.experimental.pallas import tpu_sc as plsc

B = 16384
D = 128
EPS = 1e-5


try:
    _info = plsc.get_sparse_core_info()
    _NC = _info.num_cores
    _NS = _info.num_subcores
except Exception:
    _NC, _NS = 2, 16
_NW = _NC * _NS
_BPW = B // _NW
_CH = 128
_NCH = _BPW // _CH


def _sc_gather2(uidx, iidx, t0, t1):
    mesh = plsc.VectorSubcoreMesh(core_axis_name="c", subcore_axis_name="s")
    f32 = jnp.float32

    @functools.partial(
        pl.kernel,
        mesh=mesh,
        out_type=[jax.ShapeDtypeStruct((B, D), f32) for _ in range(2)],
        scratch_types=(
            [pltpu.VMEM((_CH,), jnp.int32) for _ in range(4)]
            + [pltpu.VMEM((_CH, D), f32) for _ in range(4)]
            + [pltpu.SemaphoreType.DMA for _ in range(4)]
        ),
    )
    def gather_k(uidx_h, iidx_h, t0_h, t1_h,
                 o0_h, o1_h,
                 uv0, uv1, iv0, iv1,
                 b00, b10, b01, b11,
                 g0, g1, s0, s1):
        uv = (uv0, uv1)
        iv = (iv0, iv1)
        ubuf = (b00, b01)
        ibuf = (b10, b11)
        gsem = (g0, g1)
        ssem = (s0, s1)
        wid = lax.axis_index("s") * _NC + lax.axis_index("c")
        base = wid * _BPW

        jobs = [(t0_h, t1_h, o0_h, o1_h, c) for c in range(_NCH)]
        nj = len(jobs)

        gh = [None] * nj
        sh = [None] * nj
        pltpu.sync_copy(uidx_h.at[pl.ds(base + jobs[0][4] * _CH, _CH)], uv[0])
        pltpu.sync_copy(iidx_h.at[pl.ds(base + jobs[0][4] * _CH, _CH)], iv[0])
        for j in range(nj):
            p = j % 2
            tu, ti, ou, oi, c = jobs[j]
            if j >= 2:
                sh[j - 2][0].wait()
                sh[j - 2][1].wait()
            gh[j] = (pltpu.async_copy(tu.at[uv[p]], ubuf[p], gsem[p]),
                     pltpu.async_copy(ti.at[iv[p]], ibuf[p], gsem[p]))
            if j >= 1:
                q = 1 - p
                tu_p, ti_p, ou_p, oi_p, c_p = jobs[j - 1]
                off_p = base + c_p * _CH
                gh[j - 1][0].wait()
                gh[j - 1][1].wait()
                sh[j - 1] = (
                    pltpu.async_copy(ubuf[q], ou_p.at[pl.ds(off_p, _CH)],
                                     ssem[q]),
                    pltpu.async_copy(ibuf[q], oi_p.at[pl.ds(off_p, _CH)],
                                     ssem[q]),
                )
            if j + 1 < nj:
                off_n = base + jobs[j + 1][4] * _CH
                pltpu.sync_copy(uidx_h.at[pl.ds(off_n, _CH)], uv[1 - p])
                pltpu.sync_copy(iidx_h.at[pl.ds(off_n, _CH)], iv[1 - p])
        j = nj - 1
        p = j % 2
        gh[j][0].wait()
        gh[j][1].wait()
        tu_p, ti_p, ou_p, oi_p, c_p = jobs[j]
        off_p = base + c_p * _CH
        sh[j] = (pltpu.async_copy(ubuf[p], ou_p.at[pl.ds(off_p, _CH)],
                                  ssem[p]),
                 pltpu.async_copy(ibuf[p], oi_p.at[pl.ds(off_p, _CH)],
                                  ssem[p]))
        sh[j - 1][0].wait()
        sh[j - 1][1].wait()
        sh[j][0].wait()
        sh[j][1].wait()

    return gather_k(uidx, iidx, t0, t1)


def _sc_gather2_cat(uidx, iidx, t0, t1):
    mesh = plsc.VectorSubcoreMesh(core_axis_name="c", subcore_axis_name="s")
    f32 = jnp.float32

    @functools.partial(
        pl.kernel,
        mesh=mesh,
        out_type=jax.ShapeDtypeStruct((B, 2 * D), f32),
        scratch_types=(
            [pltpu.VMEM((_CH,), jnp.int32) for _ in range(4)]
            + [pltpu.VMEM((_CH, D), f32) for _ in range(4)]
            + [pltpu.SemaphoreType.DMA for _ in range(4)]
        ),
    )
    def gather_k(uidx_h, iidx_h, t0_h, t1_h, o_h,
                 uv0, uv1, iv0, iv1,
                 b00, b10, b01, b11,
                 g0, g1, s0, s1):
        uv = (uv0, uv1)
        iv = (iv0, iv1)
        ubuf = (b00, b01)
        ibuf = (b10, b11)
        gsem = (g0, g1)
        ssem = (s0, s1)
        wid = lax.axis_index("s") * _NC + lax.axis_index("c")
        base = wid * _BPW

        def scat(j, p):
            off = base + j * _CH
            return (
                pltpu.async_copy(ubuf[p],
                                 o_h.at[pl.ds(off, _CH), pl.ds(0, D)],
                                 ssem[p]),
                pltpu.async_copy(ibuf[p],
                                 o_h.at[pl.ds(off, _CH), pl.ds(D, D)],
                                 ssem[p]),
            )

        gh = [None] * _NCH
        sh = [None] * _NCH
        pltpu.sync_copy(uidx_h.at[pl.ds(base, _CH)], uv[0])
        pltpu.sync_copy(iidx_h.at[pl.ds(base, _CH)], iv[0])
        for j in range(_NCH):
            p = j % 2
            if j >= 2:
                sh[j - 2][0].wait()
                sh[j - 2][1].wait()
            gh[j] = (pltpu.async_copy(t0_h.at[uv[p]], ubuf[p], gsem[p]),
                     pltpu.async_copy(t1_h.at[iv[p]], ibuf[p], gsem[p]))
            if j >= 1:
                gh[j - 1][0].wait()
                gh[j - 1][1].wait()
                sh[j - 1] = scat(j - 1, 1 - p)
            if j + 1 < _NCH:
                off_n = base + (j + 1) * _CH
                pltpu.sync_copy(uidx_h.at[pl.ds(off_n, _CH)], uv[1 - p])
                pltpu.sync_copy(iidx_h.at[pl.ds(off_n, _CH)], iv[1 - p])
        j = _NCH - 1
        p = j % 2
        gh[j][0].wait()
        gh[j][1].wait()
        sh[j] = scat(j, p)
        sh[j - 1][0].wait()
        sh[j - 1][1].wait()
        sh[j][0].wait()
        sh[j][1].wait()

    return gather_k(uidx, iidx, t0, t1)


_BLK = 2048
_NB = B // _BLK


def _leaky(z):
    return jnp.maximum(z, 0.1 * z)


def _accum_stats(a, st_ref):
    ps = jnp.stack([jnp.sum(a, axis=0), jnp.sum(a * a, axis=0)])

    @pl.when(pl.program_id(0) == 0)
    def _():
        st_ref[...] = ps

    @pl.when(pl.program_id(0) > 0)
    def _():
        st_ref[...] = st_ref[...] + ps


def _norm_params(st, g, be):
    m = st[0] * (1.0 / B)
    var = st[1] * (1.0 / B) - m * m
    scale = g * lax.rsqrt(var + EPS)
    shift = be - m * scale
    return scale, shift


def _stats_vpu(a, st_ref):
    ps = jnp.stack([jnp.sum(a, axis=0), jnp.sum(a * a, axis=0)])

    @pl.when(pl.program_id(1) == 0)
    def _():
        st_ref[...] = ps

    @pl.when(pl.program_id(1) > 0)
    def _():
        st_ref[...] = st_ref[...] + ps


def _mega_body(xall_ref, ug_ref, ig_ref,
               w1_ref, b1_ref, g1_ref, be1_ref,
               w2_ref, b2_ref, g2_ref, be2_ref,
               w3_ref, b3_ref, g3_ref, be3_ref,
               wo_ref, bo_ref,
               o_ref,
               h1s, h2s, h3s, st1, st2, st3,
               w2p, b2p, w3p, b3p, wbp, csp):
    s = pl.program_id(0)
    i = pl.program_id(1)
    rows = pl.ds(i * _BLK, _BLK)
    bf16 = jnp.bfloat16
    f32 = jnp.float32

    @pl.when(s == 0)
    def _():
        w = w1_ref[...].astype(bf16)
        z = (jnp.dot(xall_ref[...].astype(bf16), w,
                     preferred_element_type=f32)
             + b1_ref[...])
        a = _leaky(z)
        ab = a.astype(bf16)
        h1s[rows, :] = ab
        _stats_vpu(a, st1)

    @pl.when(s == 1)
    def _():
        @pl.when(i == 0)
        def _():
            scale, shift = _norm_params(st1[...], g1_ref[...], be1_ref[...])
            wf = w2_ref[...]
            w2p[...] = (scale[:, None] * wf).astype(bf16)
            b2p[...] = (jnp.dot(shift[None, :], wf,
                                preferred_element_type=f32)
                        + b2_ref[...][None, :])

        z = (jnp.dot(h1s[rows, :], w2p[...], preferred_element_type=f32)
             + b2p[...])
        a = _leaky(z)
        ab = a.astype(bf16)
        h2s[rows, :] = ab
        _stats_vpu(a, st2)

    @pl.when(s == 2)
    def _():
        @pl.when(i == 0)
        def _():
            scale, shift = _norm_params(st2[...], g2_ref[...], be2_ref[...])
            wf = w3_ref[...]
            w3p[...] = (scale[:, None] * wf).astype(bf16)
            b3p[...] = (jnp.dot(shift[None, :], wf,
                                preferred_element_type=f32)
                        + b3_ref[...][None, :])

        z = (jnp.dot(h2s[rows, :], w3p[...], preferred_element_type=f32)
             + b3p[...])
        a = _leaky(z)
        ab = a.astype(bf16)
        h3s[rows, :] = ab
        _stats_vpu(a, st3)

    @pl.when(s == 3)
    def _():
        wo = wo_ref[...][:, 0]

        @pl.when(i == 0)
        def _():
            scale, shift = _norm_params(st3[...], g3_ref[...], be3_ref[...])
            wbp[...] = (scale * wo[D:])[None, :]
            csp[...] = (shift * wo[D:])[None, :]

        gmf = ug_ref[...] * ig_ref[...]
        r = (jnp.sum(gmf * wo[:D] + h3s[rows, :].astype(f32) * wbp[...],
                     axis=1)
             + (jnp.sum(csp[...]) + bo_ref[0]))
        o_ref[...] = jax.nn.sigmoid(r)


def kernel(user_indices, item_indices, ue_gmf, ie_gmf, ue_mlp, ie_mlp,
           W1, b1, g1, be1, W2, b2, g2, be2, W3, b3, g3, be3, Wo, bo):
    uidx = user_indices.astype(jnp.int32)
    iidx = item_indices.astype(jnp.int32)

    xall = _sc_gather2_cat(uidx, iidx, ue_mlp, ie_mlp)
    ug, ig = _sc_gather2(uidx, iidx, ue_gmf, ie_gmf)

    f32 = jnp.float32
    bf16 = jnp.bfloat16

    def stage0_rows(h):
        return pl.BlockSpec(
            (_BLK, h), lambda s, i: (jnp.where(s == 0, i, 0), 0))

    def stage3_rows(h):
        return pl.BlockSpec(
            (_BLK, h), lambda s, i: (jnp.where(s == 3, i, 0), 0))

    def const2():
        return pl.BlockSpec(None, lambda s, i: (0, 0))

    def const1():
        return pl.BlockSpec(None, lambda s, i: (0,))

    out = pl.pallas_call(
        _mega_body,
        grid=(4, _NB),
        in_specs=[stage0_rows(2 * D),
                  stage3_rows(D), stage3_rows(D),
                  const2(), const1(), const1(), const1(),
                  const2(), const1(), const1(), const1(),
                  const2(), const1(), const1(), const1(),
                  const2(), const1()],
        out_specs=pl.BlockSpec((_BLK,), lambda s, i: (jnp.where(s == 3, i, 0),)),
        out_shape=jax.ShapeDtypeStruct((B,), f32),
        scratch_shapes=[
            pltpu.VMEM((B, 512), bf16),
            pltpu.VMEM((B, 256), bf16),
            pltpu.VMEM((B, 128), bf16),
            pltpu.VMEM((2, 512), f32),
            pltpu.VMEM((2, 256), f32),
            pltpu.VMEM((2, 128), f32),
            pltpu.VMEM((512, 256), bf16),
            pltpu.VMEM((1, 256), f32),
            pltpu.VMEM((256, 128), bf16),
            pltpu.VMEM((1, 128), f32),
            pltpu.VMEM((1, 128), f32),
            pltpu.VMEM((1, 128), f32),
        ],
    )(xall, ug, ig,
      W1, b1, g1, be1,
      W2, b2, g2, be2,
      W3, b3, g3, be3,
      Wo, bo)

    return out


# speedup vs baseline: 1.1404x; 1.0130x over previous
import functools

import jax
import jax.numpy as jnp
from jax import lax
from jax.experimental import pallas as pl
from jax.experimental.pallas import tpu as pltpu
from jax.experimental.pallas import tpu_sc as plsc

B = 16384
D = 128
EPS = 1e-5


try:
    _info = plsc.get_sparse_core_info()
    _NC = _info.num_cores
    _NS = _info.num_subcores
except Exception:
    _NC, _NS = 2, 16
_NW = _NC * _NS
_BPW = B // _NW
_CH = 128
_NCH = _BPW // _CH


def _sc_gather2_cat(uidx, iidx, t0, t1):
    mesh = plsc.VectorSubcoreMesh(core_axis_name="c", subcore_axis_name="s")
    f32 = jnp.float32

    @functools.partial(
        pl.kernel,
        mesh=mesh,
        out_type=jax.ShapeDtypeStruct((B, 2 * D), f32),
        scratch_types=(
            [pltpu.VMEM((_CH,), jnp.int32) for _ in range(4)]
            + [pltpu.VMEM((_CH, D), f32) for _ in range(4)]
            + [pltpu.SemaphoreType.DMA for _ in range(4)]
        ),
    )
    def gather_k(uidx_h, iidx_h, t0_h, t1_h, o_h,
                 uv0, uv1, iv0, iv1,
                 b00, b10, b01, b11,
                 g0, g1, s0, s1):
        uv = (uv0, uv1)
        iv = (iv0, iv1)
        ubuf = (b00, b01)
        ibuf = (b10, b11)
        gsem = (g0, g1)
        ssem = (s0, s1)
        wid = lax.axis_index("s") * _NC + lax.axis_index("c")
        base = wid * _BPW

        def scat(j, p):
            off = base + j * _CH
            return (
                pltpu.async_copy(ubuf[p],
                                 o_h.at[pl.ds(off, _CH), pl.ds(0, D)],
                                 ssem[p]),
                pltpu.async_copy(ibuf[p],
                                 o_h.at[pl.ds(off, _CH), pl.ds(D, D)],
                                 ssem[p]),
            )

        gh = [None] * _NCH
        sh = [None] * _NCH
        pltpu.sync_copy(uidx_h.at[pl.ds(base, _CH)], uv[0])
        pltpu.sync_copy(iidx_h.at[pl.ds(base, _CH)], iv[0])
        for j in range(_NCH):
            p = j % 2
            if j >= 2:
                sh[j - 2][0].wait()
                sh[j - 2][1].wait()
            gh[j] = (pltpu.async_copy(t0_h.at[uv[p]], ubuf[p], gsem[p]),
                     pltpu.async_copy(t1_h.at[iv[p]], ibuf[p], gsem[p]))
            if j >= 1:
                gh[j - 1][0].wait()
                gh[j - 1][1].wait()
                sh[j - 1] = scat(j - 1, 1 - p)
            if j + 1 < _NCH:
                off_n = base + (j + 1) * _CH
                pltpu.sync_copy(uidx_h.at[pl.ds(off_n, _CH)], uv[1 - p])
                pltpu.sync_copy(iidx_h.at[pl.ds(off_n, _CH)], iv[1 - p])
        j = _NCH - 1
        p = j % 2
        gh[j][0].wait()
        gh[j][1].wait()
        sh[j] = scat(j, p)
        sh[j - 1][0].wait()
        sh[j - 1][1].wait()
        sh[j][0].wait()
        sh[j][1].wait()

    return gather_k(uidx, iidx, t0, t1)


_BLK = 2048
_NB = B // _BLK


def _leaky(z):
    return jnp.maximum(z, 0.1 * z)


def _accum_stats(a, st_ref):
    ps = jnp.stack([jnp.sum(a, axis=0), jnp.sum(a * a, axis=0)])

    @pl.when(pl.program_id(0) == 0)
    def _():
        st_ref[...] = ps

    @pl.when(pl.program_id(0) > 0)
    def _():
        st_ref[...] = st_ref[...] + ps


def _norm_params(st, g, be):
    m = st[0] * (1.0 / B)
    var = st[1] * (1.0 / B) - m * m
    scale = g * lax.rsqrt(var + EPS)
    shift = be - m * scale
    return scale, shift


def _stats_vpu(a, st_ref):
    ps = jnp.stack([jnp.sum(a, axis=0), jnp.sum(a * a, axis=0)])

    @pl.when(pl.program_id(1) == 0)
    def _():
        st_ref[...] = ps

    @pl.when(pl.program_id(1) > 0)
    def _():
        st_ref[...] = st_ref[...] + ps


def _mega_body(xall_ref, gcat_ref,
               w1_ref, b1_ref, g1_ref, be1_ref,
               w2_ref, b2_ref, g2_ref, be2_ref,
               w3_ref, b3_ref, g3_ref, be3_ref,
               wo_ref, bo_ref,
               o_ref,
               h1s, h2s, h3s, st1, st2, st3,
               w2p, b2p, w3p, b3p, wav, wbv, csc):
    s = pl.program_id(0)
    i = pl.program_id(1)
    rows = pl.ds(i * _BLK, _BLK)
    bf16 = jnp.bfloat16
    f32 = jnp.float32

    @pl.when(s == 0)
    def _():
        w = w1_ref[...].astype(bf16)
        z = (jnp.dot(xall_ref[...].astype(bf16), w,
                     preferred_element_type=f32)
             + b1_ref[...])
        a = _leaky(z)
        ab = a.astype(bf16)
        h1s[rows, :] = ab
        _stats_vpu(a, st1)

    @pl.when(s == 1)
    def _():
        @pl.when(i == 0)
        def _():
            scale, shift = _norm_params(st1[...], g1_ref[...], be1_ref[...])
            wf = w2_ref[...]
            w2p[...] = (scale[:, None] * wf).astype(bf16)
            b2p[...] = (jnp.dot(shift[None, :], wf,
                                preferred_element_type=f32)
                        + b2_ref[...][None, :])

        z = (jnp.dot(h1s[rows, :], w2p[...], preferred_element_type=f32)
             + b2p[...])
        a = _leaky(z)
        ab = a.astype(bf16)
        h2s[rows, :] = ab
        _stats_vpu(a, st2)

    @pl.when(s == 2)
    def _():
        @pl.when(i == 0)
        def _():
            scale, shift = _norm_params(st2[...], g2_ref[...], be2_ref[...])
            wf = w3_ref[...]
            w3p[...] = (scale[:, None] * wf).astype(bf16)
            b3p[...] = (jnp.dot(shift[None, :], wf,
                                preferred_element_type=f32)
                        + b3_ref[...][None, :])

        z = (jnp.dot(h2s[rows, :], w3p[...], preferred_element_type=f32)
             + b3p[...])
        a = _leaky(z)
        ab = a.astype(bf16)
        h3s[rows, :] = ab
        _stats_vpu(a, st3)

    @pl.when(s == 3)
    def _():
        wo = wo_ref[...]

        @pl.when(i == 0)
        def _():
            scale, shift = _norm_params(st3[...], g3_ref[...], be3_ref[...])
            wav[...] = wo[:D].astype(bf16)
            wbv[...] = (scale[:, None] * wo[D:]).astype(bf16)
            csc[0, 0] = jnp.sum(shift[:, None] * wo[D:]) + bo_ref[0]

        g2 = gcat_ref[...]
        gmf = (g2[:, :D] * g2[:, D:]).astype(bf16)
        r = (jnp.dot(gmf, wav[...], preferred_element_type=f32)
             + jnp.dot(h3s[rows, :], wbv[...], preferred_element_type=f32)
             )[:, 0] + csc[0, 0]
        o_ref[...] = jax.nn.sigmoid(r)


def kernel(user_indices, item_indices, ue_gmf, ie_gmf, ue_mlp, ie_mlp,
           W1, b1, g1, be1, W2, b2, g2, be2, W3, b3, g3, be3, Wo, bo):
    uidx = user_indices.astype(jnp.int32)
    iidx = item_indices.astype(jnp.int32)

    xall = _sc_gather2_cat(uidx, iidx, ue_mlp, ie_mlp)
    gcat = _sc_gather2_cat(uidx, iidx, ue_gmf, ie_gmf)

    f32 = jnp.float32
    bf16 = jnp.bfloat16

    def stage0_rows(h):
        return pl.BlockSpec(
            (_BLK, h), lambda s, i: (jnp.where(s == 0, i, 0), 0))

    def stage3_rows(h):
        return pl.BlockSpec(
            (_BLK, h), lambda s, i: (jnp.where(s == 3, i, 0), 0))

    def const2():
        return pl.BlockSpec(None, lambda s, i: (0, 0))

    def const1():
        return pl.BlockSpec(None, lambda s, i: (0,))

    out = pl.pallas_call(
        _mega_body,
        grid=(4, _NB),
        in_specs=[stage0_rows(2 * D),
                  stage3_rows(2 * D),
                  const2(), const1(), const1(), const1(),
                  const2(), const1(), const1(), const1(),
                  const2(), const1(), const1(), const1(),
                  const2(), const1()],
        out_specs=pl.BlockSpec((_BLK,), lambda s, i: (jnp.where(s == 3, i, 0),)),
        out_shape=jax.ShapeDtypeStruct((B,), f32),
        scratch_shapes=[
            pltpu.VMEM((B, 512), bf16),
            pltpu.VMEM((B, 256), bf16),
            pltpu.VMEM((B, 128), bf16),
            pltpu.VMEM((2, 512), f32),
            pltpu.VMEM((2, 256), f32),
            pltpu.VMEM((2, 128), f32),
            pltpu.VMEM((512, 256), bf16),
            pltpu.VMEM((1, 256), f32),
            pltpu.VMEM((256, 128), bf16),
            pltpu.VMEM((1, 128), f32),
            pltpu.VMEM((D, 1), bf16),
            pltpu.VMEM((D, 1), bf16),
            pltpu.SMEM((1, 1), f32),
        ],
    )(xall, gcat,
      W1, b1, g1, be1,
      W2, b2, g2, be2,
      W3, b3, g3, be3,
      Wo, bo)

    return out


# speedup vs baseline: 1.2165x; 1.0668x over previous
import functools

import jax
import jax.numpy as jnp
from jax import lax
from jax.experimental import pallas as pl
from jax.experimental.pallas import tpu as pltpu
from jax.experimental.pallas import tpu_sc as plsc

B = 16384
D = 128
EPS = 1e-5


try:
    _info = plsc.get_sparse_core_info()
    _NC = _info.num_cores
    _NS = _info.num_subcores
except Exception:
    _NC, _NS = 2, 16
_NW = _NC * _NS
_BPW = B // _NW
_CH = 128
_NCH = _BPW // _CH


def _sc_gather2_cat(uidx, iidx, t0, t1):
    mesh = plsc.VectorSubcoreMesh(core_axis_name="c", subcore_axis_name="s")
    f32 = jnp.float32

    @functools.partial(
        pl.kernel,
        mesh=mesh,
        out_type=jax.ShapeDtypeStruct((B, 2 * D), f32),
        scratch_types=(
            [pltpu.VMEM((_CH,), jnp.int32) for _ in range(4)]
            + [pltpu.VMEM((_CH, D), f32) for _ in range(4)]
            + [pltpu.SemaphoreType.DMA for _ in range(4)]
        ),
    )
    def gather_k(uidx_h, iidx_h, t0_h, t1_h, o_h,
                 uv0, uv1, iv0, iv1,
                 b00, b10, b01, b11,
                 g0, g1, s0, s1):
        uv = (uv0, uv1)
        iv = (iv0, iv1)
        ubuf = (b00, b01)
        ibuf = (b10, b11)
        gsem = (g0, g1)
        ssem = (s0, s1)
        wid = lax.axis_index("s") * _NC + lax.axis_index("c")
        base = wid * _BPW

        def scat(j, p):
            off = base + j * _CH
            return (
                pltpu.async_copy(ubuf[p],
                                 o_h.at[pl.ds(off, _CH), pl.ds(0, D)],
                                 ssem[p]),
                pltpu.async_copy(ibuf[p],
                                 o_h.at[pl.ds(off, _CH), pl.ds(D, D)],
                                 ssem[p]),
            )

        gh = [None] * _NCH
        sh = [None] * _NCH
        pltpu.sync_copy(uidx_h.at[pl.ds(base, _CH)], uv[0])
        pltpu.sync_copy(iidx_h.at[pl.ds(base, _CH)], iv[0])
        for j in range(_NCH):
            p = j % 2
            if j >= 2:
                sh[j - 2][0].wait()
                sh[j - 2][1].wait()
            gh[j] = (pltpu.async_copy(t0_h.at[uv[p]], ubuf[p], gsem[p]),
                     pltpu.async_copy(t1_h.at[iv[p]], ibuf[p], gsem[p]))
            if j >= 1:
                gh[j - 1][0].wait()
                gh[j - 1][1].wait()
                sh[j - 1] = scat(j - 1, 1 - p)
            if j + 1 < _NCH:
                off_n = base + (j + 1) * _CH
                pltpu.sync_copy(uidx_h.at[pl.ds(off_n, _CH)], uv[1 - p])
                pltpu.sync_copy(iidx_h.at[pl.ds(off_n, _CH)], iv[1 - p])
        j = _NCH - 1
        p = j % 2
        gh[j][0].wait()
        gh[j][1].wait()
        sh[j] = scat(j, p)
        sh[j - 1][0].wait()
        sh[j - 1][1].wait()
        sh[j][0].wait()
        sh[j][1].wait()

    return gather_k(uidx, iidx, t0, t1)


_BLK = 4096
_NB = B // _BLK


def _leaky(z):
    return jnp.maximum(z, 0.1 * z)


def _accum_stats(a, st_ref):
    ps = jnp.stack([jnp.sum(a, axis=0), jnp.sum(a * a, axis=0)])

    @pl.when(pl.program_id(0) == 0)
    def _():
        st_ref[...] = ps

    @pl.when(pl.program_id(0) > 0)
    def _():
        st_ref[...] = st_ref[...] + ps


def _norm_params(st, g, be):
    m = st[0] * (1.0 / B)
    var = st[1] * (1.0 / B) - m * m
    scale = g * lax.rsqrt(var + EPS)
    shift = be - m * scale
    return scale, shift


def _stats_vpu(a, st_ref):
    ps = jnp.stack([jnp.sum(a, axis=0), jnp.sum(a * a, axis=0)])

    @pl.when(pl.program_id(1) == 0)
    def _():
        st_ref[...] = ps

    @pl.when(pl.program_id(1) > 0)
    def _():
        st_ref[...] = st_ref[...] + ps


def _mega_body(xall_ref, gcat_ref,
               w1_ref, b1_ref, g1_ref, be1_ref,
               w2_ref, b2_ref, g2_ref, be2_ref,
               w3_ref, b3_ref, g3_ref, be3_ref,
               wo_ref, bo_ref,
               o_ref,
               h1s, h2s, h3s, st1, st2, st3,
               w2p, b2p, w3p, b3p, wav, wbv, csc):
    s = pl.program_id(0)
    i = pl.program_id(1)
    rows = pl.ds(i * _BLK, _BLK)
    bf16 = jnp.bfloat16
    f32 = jnp.float32

    @pl.when(s == 0)
    def _():
        w = w1_ref[...].astype(bf16)
        z = (jnp.dot(xall_ref[...].astype(bf16), w,
                     preferred_element_type=f32)
             + b1_ref[...])
        a = _leaky(z)
        ab = a.astype(bf16)
        h1s[rows, :] = ab
        _stats_vpu(a, st1)

    @pl.when(s == 1)
    def _():
        @pl.when(i == 0)
        def _():
            scale, shift = _norm_params(st1[...], g1_ref[...], be1_ref[...])
            wf = w2_ref[...]
            w2p[...] = (scale[:, None] * wf).astype(bf16)
            b2p[...] = (jnp.dot(shift[None, :], wf,
                                preferred_element_type=f32)
                        + b2_ref[...][None, :])

        z = (jnp.dot(h1s[rows, :], w2p[...], preferred_element_type=f32)
             + b2p[...])
        a = _leaky(z)
        ab = a.astype(bf16)
        h2s[rows, :] = ab
        _stats_vpu(a, st2)

    @pl.when(s == 2)
    def _():
        @pl.when(i == 0)
        def _():
            scale, shift = _norm_params(st2[...], g2_ref[...], be2_ref[...])
            wf = w3_ref[...]
            w3p[...] = (scale[:, None] * wf).astype(bf16)
            b3p[...] = (jnp.dot(shift[None, :], wf,
                                preferred_element_type=f32)
                        + b3_ref[...][None, :])

        z = (jnp.dot(h2s[rows, :], w3p[...], preferred_element_type=f32)
             + b3p[...])
        a = _leaky(z)
        ab = a.astype(bf16)
        h3s[rows, :] = ab
        _stats_vpu(a, st3)

    @pl.when(s == 3)
    def _():
        wo = wo_ref[...]

        @pl.when(i == 0)
        def _():
            scale, shift = _norm_params(st3[...], g3_ref[...], be3_ref[...])
            wav[...] = wo[:D].astype(bf16)
            wbv[...] = (scale[:, None] * wo[D:]).astype(bf16)
            csc[0, 0] = jnp.sum(shift[:, None] * wo[D:]) + bo_ref[0]

        g2 = gcat_ref[...]
        gmf = (g2[:, :D] * g2[:, D:]).astype(bf16)
        r = (jnp.dot(gmf, wav[...], preferred_element_type=f32)
             + jnp.dot(h3s[rows, :], wbv[...], preferred_element_type=f32)
             )[:, 0] + csc[0, 0]
        o_ref[...] = jax.nn.sigmoid(r)


def kernel(user_indices, item_indices, ue_gmf, ie_gmf, ue_mlp, ie_mlp,
           W1, b1, g1, be1, W2, b2, g2, be2, W3, b3, g3, be3, Wo, bo):
    uidx = user_indices.astype(jnp.int32)
    iidx = item_indices.astype(jnp.int32)

    xall = _sc_gather2_cat(uidx, iidx, ue_mlp, ie_mlp)
    gcat = _sc_gather2_cat(uidx, iidx, ue_gmf, ie_gmf)

    f32 = jnp.float32
    bf16 = jnp.bfloat16

    def stage0_rows(h):
        return pl.BlockSpec(
            (_BLK, h), lambda s, i: (jnp.where(s == 0, i, 0), 0))

    def stage3_rows(h):
        return pl.BlockSpec(
            (_BLK, h), lambda s, i: (jnp.where(s == 3, i, 0), 0))

    def const2():
        return pl.BlockSpec(None, lambda s, i: (0, 0))

    def const1():
        return pl.BlockSpec(None, lambda s, i: (0,))

    out = pl.pallas_call(
        _mega_body,
        grid=(4, _NB),
        in_specs=[stage0_rows(2 * D),
                  stage3_rows(2 * D),
                  const2(), const1(), const1(), const1(),
                  const2(), const1(), const1(), const1(),
                  const2(), const1(), const1(), const1(),
                  const2(), const1()],
        out_specs=pl.BlockSpec((_BLK,), lambda s, i: (jnp.where(s == 3, i, 0),)),
        out_shape=jax.ShapeDtypeStruct((B,), f32),
        scratch_shapes=[
            pltpu.VMEM((B, 512), bf16),
            pltpu.VMEM((B, 256), bf16),
            pltpu.VMEM((B, 128), bf16),
            pltpu.VMEM((2, 512), f32),
            pltpu.VMEM((2, 256), f32),
            pltpu.VMEM((2, 128), f32),
            pltpu.VMEM((512, 256), bf16),
            pltpu.VMEM((1, 256), f32),
            pltpu.VMEM((256, 128), bf16),
            pltpu.VMEM((1, 128), f32),
            pltpu.VMEM((D, 1), bf16),
            pltpu.VMEM((D, 1), bf16),
            pltpu.SMEM((1, 1), f32),
        ],
    )(xall, gcat,
      W1, b1, g1, be1,
      W2, b2, g2, be2,
      W3, b3, g3, be3,
      Wo, bo)

    return out


# speedup vs baseline: 1.4084x; 1.1577x over previous
import functools

import jax
import jax.numpy as jnp
from jax import lax
from jax.experimental import pallas as pl
from jax.experimental.pallas import tpu as pltpu
from jax.experimental.pallas import tpu_sc as plsc

B = 16384
D = 128
EPS = 1e-5


try:
    _info = plsc.get_sparse_core_info()
    _NC = _info.num_cores
    _NS = _info.num_subcores
except Exception:
    _NC, _NS = 2, 16
_NW = _NC * _NS
_BPW = B // _NW
_CH = 128
_NCH = _BPW // _CH


def _sc_gather2_cat(uidx, iidx, t0, t1):
    mesh = plsc.VectorSubcoreMesh(core_axis_name="c", subcore_axis_name="s")
    f32 = jnp.float32

    @functools.partial(
        pl.kernel,
        mesh=mesh,
        out_type=jax.ShapeDtypeStruct((B, 2 * D), f32),
        scratch_types=(
            [pltpu.VMEM((_CH,), jnp.int32) for _ in range(4)]
            + [pltpu.VMEM((_CH, D), f32) for _ in range(4)]
            + [pltpu.SemaphoreType.DMA for _ in range(4)]
        ),
    )
    def gather_k(uidx_h, iidx_h, t0_h, t1_h, o_h,
                 uv0, uv1, iv0, iv1,
                 b00, b10, b01, b11,
                 g0, g1, s0, s1):
        uv = (uv0, uv1)
        iv = (iv0, iv1)
        ubuf = (b00, b01)
        ibuf = (b10, b11)
        gsem = (g0, g1)
        ssem = (s0, s1)
        wid = lax.axis_index("s") * _NC + lax.axis_index("c")
        base = wid * _BPW

        def scat(j, p):
            off = base + j * _CH
            return (
                pltpu.async_copy(ubuf[p],
                                 o_h.at[pl.ds(off, _CH), pl.ds(0, D)],
                                 ssem[p]),
                pltpu.async_copy(ibuf[p],
                                 o_h.at[pl.ds(off, _CH), pl.ds(D, D)],
                                 ssem[p]),
            )

        gh = [None] * _NCH
        sh = [None] * _NCH
        pltpu.sync_copy(uidx_h.at[pl.ds(base, _CH)], uv[0])
        pltpu.sync_copy(iidx_h.at[pl.ds(base, _CH)], iv[0])
        for j in range(_NCH):
            p = j % 2
            if j >= 2:
                sh[j - 2][0].wait()
                sh[j - 2][1].wait()
            gh[j] = (pltpu.async_copy(t0_h.at[uv[p]], ubuf[p], gsem[p]),
                     pltpu.async_copy(t1_h.at[iv[p]], ibuf[p], gsem[p]))
            if j >= 1:
                gh[j - 1][0].wait()
                gh[j - 1][1].wait()
                sh[j - 1] = scat(j - 1, 1 - p)
            if j + 1 < _NCH:
                off_n = base + (j + 1) * _CH
                pltpu.sync_copy(uidx_h.at[pl.ds(off_n, _CH)], uv[1 - p])
                pltpu.sync_copy(iidx_h.at[pl.ds(off_n, _CH)], iv[1 - p])
        j = _NCH - 1
        p = j % 2
        gh[j][0].wait()
        gh[j][1].wait()
        sh[j] = scat(j, p)
        sh[j - 1][0].wait()
        sh[j - 1][1].wait()
        sh[j][0].wait()
        sh[j][1].wait()

    return gather_k(uidx, iidx, t0, t1)


_BLK = 4096
_NB = B // _BLK


def _leaky(z):
    return jnp.maximum(z, 0.1 * z)


def _accum_stats(a, st_ref):
    ps = jnp.stack([jnp.sum(a, axis=0), jnp.sum(a * a, axis=0)])

    @pl.when(pl.program_id(0) == 0)
    def _():
        st_ref[...] = ps

    @pl.when(pl.program_id(0) > 0)
    def _():
        st_ref[...] = st_ref[...] + ps


def _norm_params(st, g, be):
    m = st[0] * (1.0 / B)
    var = st[1] * (1.0 / B) - m * m
    scale = g * lax.rsqrt(var + EPS)
    shift = be - m * scale
    return scale, shift


def _stats_vpu(a, st_ref):
    ps = jnp.stack([jnp.sum(a, axis=0), jnp.sum(a * a, axis=0)])

    @pl.when(pl.program_id(1) == 0)
    def _():
        st_ref[...] = ps

    @pl.when(pl.program_id(1) > 0)
    def _():
        st_ref[...] = st_ref[...] + ps


def _mega_body(xall_ref,
               w1_ref, b1_ref, g1_ref, be1_ref,
               w2_ref, b2_ref, g2_ref, be2_ref,
               w3_ref, b3_ref,
               h3o_ref, st3o_ref,
               h1s, h2s, st1, st2,
               w2p, b2p, w3p, b3p):
    s = pl.program_id(0)
    i = pl.program_id(1)
    rows = pl.ds(i * _BLK, _BLK)
    bf16 = jnp.bfloat16
    f32 = jnp.float32

    @pl.when(s == 0)
    def _():
        w = w1_ref[...].astype(bf16)
        z = (jnp.dot(xall_ref[...].astype(bf16), w,
                     preferred_element_type=f32)
             + b1_ref[...])
        a = _leaky(z)
        ab = a.astype(bf16)
        h1s[rows, :] = ab
        _stats_vpu(a, st1)

    @pl.when(s == 1)
    def _():
        @pl.when(i == 0)
        def _():
            scale, shift = _norm_params(st1[...], g1_ref[...], be1_ref[...])
            wf = w2_ref[...]
            w2p[...] = (scale[:, None] * wf).astype(bf16)
            b2p[...] = (jnp.dot(shift[None, :], wf,
                                preferred_element_type=f32)
                        + b2_ref[...][None, :])

        z = (jnp.dot(h1s[rows, :], w2p[...], preferred_element_type=f32)
             + b2p[...])
        a = _leaky(z)
        ab = a.astype(bf16)
        h2s[rows, :] = ab
        _stats_vpu(a, st2)

    @pl.when(s == 2)
    def _():
        @pl.when(i == 0)
        def _():
            scale, shift = _norm_params(st2[...], g2_ref[...], be2_ref[...])
            wf = w3_ref[...]
            w3p[...] = (scale[:, None] * wf).astype(bf16)
            b3p[...] = (jnp.dot(shift[None, :], wf,
                                preferred_element_type=f32)
                        + b3_ref[...][None, :])

        z = (jnp.dot(h2s[rows, :], w3p[...], preferred_element_type=f32)
             + b3p[...])
        a = _leaky(z)
        h3o_ref[...] = a.astype(bf16)
        _stats_vpu(a, st3o_ref)


def _head_body(h3_ref, st3_ref, g3_ref, be3_ref, gcat_ref, wo_ref, bo_ref,
               o_ref, wav, wbv, csc):
    i = pl.program_id(0)
    bf16 = jnp.bfloat16
    f32 = jnp.float32
    wo = wo_ref[...]

    @pl.when(i == 0)
    def _():
        scale, shift = _norm_params(st3_ref[...], g3_ref[...], be3_ref[...])
        wav[...] = wo[:D].astype(bf16)
        wbv[...] = (scale[:, None] * wo[D:]).astype(bf16)
        csc[0, 0] = jnp.sum(shift[:, None] * wo[D:]) + bo_ref[0]

    g2 = gcat_ref[...]
    gmf = (g2[:, :D] * g2[:, D:]).astype(bf16)
    r = (jnp.dot(gmf, wav[...], preferred_element_type=f32)
         + jnp.dot(h3_ref[...], wbv[...], preferred_element_type=f32)
         )[:, 0] + csc[0, 0]
    o_ref[...] = jax.nn.sigmoid(r)


def kernel(user_indices, item_indices, ue_gmf, ie_gmf, ue_mlp, ie_mlp,
           W1, b1, g1, be1, W2, b2, g2, be2, W3, b3, g3, be3, Wo, bo):
    uidx = user_indices.astype(jnp.int32)
    iidx = item_indices.astype(jnp.int32)

    xall = _sc_gather2_cat(uidx, iidx, ue_mlp, ie_mlp)

    f32 = jnp.float32
    bf16 = jnp.bfloat16

    def stage0_rows(h):
        return pl.BlockSpec(
            (_BLK, h), lambda s, i: (jnp.where(s == 0, i, 0), 0))

    def const2():
        return pl.BlockSpec(None, lambda s, i: (0, 0))

    def const1():
        return pl.BlockSpec(None, lambda s, i: (0,))

    h3, st3 = pl.pallas_call(
        _mega_body,
        grid=(3, _NB),
        in_specs=[stage0_rows(2 * D),
                  const2(), const1(), const1(), const1(),
                  const2(), const1(), const1(), const1(),
                  const2(), const1()],
        out_specs=[pl.BlockSpec((_BLK, D),
                                lambda s, i: (jnp.where(s == 2, i, 0), 0)),
                   const2()],
        out_shape=[jax.ShapeDtypeStruct((B, D), bf16),
                   jax.ShapeDtypeStruct((2, D), f32)],
        scratch_shapes=[
            pltpu.VMEM((B, 512), bf16),
            pltpu.VMEM((B, 256), bf16),
            pltpu.VMEM((2, 512), f32),
            pltpu.VMEM((2, 256), f32),
            pltpu.VMEM((512, 256), bf16),
            pltpu.VMEM((1, 256), f32),
            pltpu.VMEM((256, 128), bf16),
            pltpu.VMEM((1, 128), f32),
        ],
    )(xall,
      W1, b1, g1, be1,
      W2, b2, g2, be2,
      W3, b3)

    gcat = _sc_gather2_cat(uidx, iidx, ue_gmf, ie_gmf)

    out = pl.pallas_call(
        _head_body,
        grid=(_NB,),
        in_specs=[pl.BlockSpec((_BLK, D), lambda i: (i, 0)),
                  pl.BlockSpec(None, lambda i: (0, 0)),
                  pl.BlockSpec(None, lambda i: (0,)),
                  pl.BlockSpec(None, lambda i: (0,)),
                  pl.BlockSpec((_BLK, 2 * D), lambda i: (i, 0)),
                  pl.BlockSpec(None, lambda i: (0, 0)),
                  pl.BlockSpec(None, lambda i: (0,))],
        out_specs=pl.BlockSpec((_BLK,), lambda i: (i,)),
        out_shape=jax.ShapeDtypeStruct((B,), f32),
        scratch_shapes=[
            pltpu.VMEM((D, 1), bf16),
            pltpu.VMEM((D, 1), bf16),
            pltpu.SMEM((1, 1), f32),
        ],
    )(h3, st3, g3, be3, gcat, Wo, bo)

    return out
